# baseline (speedup 1.0000x reference)
import jax
import jax.numpy as jnp
from jax.experimental import pallas as pl
from jax.experimental.pallas import tpu as pltpu

BATCH = 1024
FEATS = 4096
BN = 256
KH = FEATS // 2
KQ = FEATS // 4


def _x_copy(x_hbm, xs_ref, sems, q):
    sl = pl.ds(q * KQ, KQ)
    return pltpu.make_async_copy(x_hbm.at[:, sl], xs_ref.at[:, sl], sems.at[q])


def _matmul_body(x_hbm, wa_ref, wb_ref, b_ref, o_ref, xs_ref, sems):
    first = pl.program_id(0) == 0

    @pl.when(first)
    def _start_fill():
        for q in range(4):
            _x_copy(x_hbm, xs_ref, sems, q).start()

    dn = (((1,), (1,)), ((), ()))

    def half_dot(w_ref, h):
        sl = pl.ds(h * KH, KH)
        return jax.lax.dot_general(
            xs_ref[:, sl], w_ref[:, sl], dimension_numbers=dn,
            preferred_element_type=jnp.float32,
            precision=jax.lax.Precision.DEFAULT,
        )

    def full_dot(w_ref):
        return jax.lax.dot_general(
            xs_ref[...], w_ref[...], dimension_numbers=dn,
            preferred_element_type=jnp.float32,
            precision=jax.lax.Precision.DEFAULT,
        )

    @pl.when(first)
    def _step0_split():
        _x_copy(x_hbm, xs_ref, sems, 0).wait()
        _x_copy(x_hbm, xs_ref, sems, 1).wait()
        acc_a = half_dot(wa_ref, 0)
        acc_b = half_dot(wb_ref, 0)
        _x_copy(x_hbm, xs_ref, sems, 2).wait()
        _x_copy(x_hbm, xs_ref, sems, 3).wait()
        acc_a = acc_a + half_dot(wa_ref, 1)
        acc_b = acc_b + half_dot(wb_ref, 1)
        o_ref[:, :BN] = acc_a + b_ref[:, :BN]
        o_ref[:, BN:] = acc_b + b_ref[:, BN:]

    @pl.when(jnp.logical_not(first))
    def _steady():
        o_ref[:, :BN] = full_dot(wa_ref) + b_ref[:, :BN]
        o_ref[:, BN:] = full_dot(wb_ref) + b_ref[:, BN:]


def kernel(x, weight, bias):
    bias2d = bias.reshape(1, FEATS)
    grid = (FEATS // (2 * BN),)
    return pl.pallas_call(
        _matmul_body,
        grid=grid,
        in_specs=[
            pl.BlockSpec(memory_space=pl.ANY),
            pl.BlockSpec((BN, FEATS), lambda j: (2 * j, 0)),
            pl.BlockSpec((BN, FEATS), lambda j: (2 * j + 1, 0)),
            pl.BlockSpec((1, 2 * BN), lambda j: (0, j)),
        ],
        out_specs=pl.BlockSpec((BATCH, 2 * BN), lambda j: (0, j)),
        out_shape=jax.ShapeDtypeStruct((BATCH, FEATS), jnp.float32),
        scratch_shapes=[
            pltpu.VMEM((BATCH, FEATS), jnp.float32),
            pltpu.SemaphoreType.DMA((4,)),
        ],
        compiler_params=pltpu.CompilerParams(
            dimension_semantics=("arbitrary",),
        ),
    )(x, weight, weight, bias2d)
